# baseline (device time: 1262653 ns/iter reference)
import jax
import jax.numpy as jnp
from jax import lax
from jax.experimental import pallas as pl
from jax.experimental.pallas import tpu as pltpu

N_DEV = 4
SQ = 2048
SKV = 2048
D_MODEL = 1024
N_HEADS = 32
HEADS_PER_DEV = 8
D_HEAD = 128
BLK = 64
SCALE = 0.08838834764831843
QT = 256
N_QT = SQ // QT
NEG = -1e9


def _ag_body(w_ref, out_ref, send_sems, recv_sems):
    my = lax.axis_index("i")
    left = lax.rem(my + N_DEV - 1, N_DEV)
    right = lax.rem(my + 1, N_DEV)

    barrier = pltpu.get_barrier_semaphore()
    for nbr in (left, right):
        pl.semaphore_signal(barrier, inc=1, device_id=(nbr,),
                            device_id_type=pl.DeviceIdType.MESH)
    pl.semaphore_wait(barrier, 2)

    out_ref[my] = w_ref[...]

    for h in range(N_DEV - 1):
        slot = lax.rem(my - h + N_DEV, N_DEV)
        rdma = pltpu.make_async_remote_copy(
            src_ref=out_ref.at[slot],
            dst_ref=out_ref.at[slot],
            send_sem=send_sems.at[h],
            recv_sem=recv_sems.at[h],
            device_id=(right,),
            device_id_type=pl.DeviceIdType.MESH,
        )
        rdma.start()
        rdma.wait()


def _attn_body(x_ref, w_ref, k_hbm, v_hbm, out_ref,
               q_ref, ctx_ref, bias_ref, kbuf, vbuf, ksem, vsem):
    g = pl.program_id(0)

    kc = pltpu.make_async_copy(k_hbm.at[:, g, :, :], kbuf, ksem)
    vc = pltpu.make_async_copy(v_hbm.at[:, g, :, :], vbuf, vsem)
    kc.start()
    vc.start()

    @pl.when(g == 0)
    def _():
        for t in range(N_QT):
            qi = lax.broadcasted_iota(jnp.int32, (QT, SKV), 0) + t * QT
            kj = lax.broadcasted_iota(jnp.int32, (QT, SKV), 1)
            qb = qi // BLK
            kb = kj // BLK
            keep = (qb == kb) | (kb == 0) | (lax.rem(qb + kb, 3) == 0)
            bias_ref[pl.ds(t * QT, QT), :] = jnp.where(
                keep, 0.0, NEG).astype(jnp.bfloat16)
        out_ref[...] = jnp.zeros((SQ, D_MODEL), jnp.float32)

    wq = w_ref[0, 0]

    def _qproj(t, _):
        r0 = t * QT
        q_ref[pl.ds(r0, QT), :] = jnp.dot(
            x_ref[pl.ds(r0, QT), :], wq,
            preferred_element_type=jnp.float32).astype(jnp.bfloat16)
        return 0

    lax.fori_loop(0, N_QT, _qproj, 0)

    kc.wait()
    vc.wait()
    for hl in range(HEADS_PER_DEV):
        kh = kbuf[:, hl, :]
        vh = vbuf[:, hl, :]
        col = hl * D_HEAD

        def _tile(t, _):
            r0 = t * QT
            qt = q_ref[pl.ds(r0, QT), col:col + D_HEAD]
            s = lax.dot_general(qt, kh, (((1,), (1,)), ((), ())),
                                preferred_element_type=jnp.float32)
            s = s * SCALE + bias_ref[pl.ds(r0, QT), :].astype(jnp.float32)
            mx = jnp.max(s, axis=1, keepdims=True)
            e = jnp.exp(s - mx)
            denom = jnp.sum(e, axis=1, keepdims=True)
            wgt = (e / denom).astype(jnp.bfloat16)
            ctx_t = lax.dot_general(wgt, vh, (((1,), (0,)), ((), ())),
                                    preferred_element_type=jnp.float32)
            ctx_ref[pl.ds(r0, QT), col:col + D_HEAD] = ctx_t.astype(
                jnp.bfloat16)
            return 0

        lax.fori_loop(0, N_QT, _tile, 0)

    wo = w_ref[0, 1]

    def _oproj(t, _):
        r0 = t * QT
        out_ref[pl.ds(r0, QT), :] += jnp.dot(
            ctx_ref[pl.ds(r0, QT), :], wo,
            preferred_element_type=jnp.float32)
        return 0

    lax.fori_loop(0, N_QT, _oproj, 0)


def kernel(x, Wq, K_ext, V_ext, Wo):
    my = lax.axis_index("i")
    xb = x[0].astype(jnp.bfloat16)
    w_cat = jnp.stack([Wq.astype(jnp.bfloat16),
                       Wo.astype(jnp.bfloat16)])
    k = lax.dynamic_index_in_dim(K_ext, my, axis=0, keepdims=False)
    v = lax.dynamic_index_in_dim(V_ext, my, axis=0, keepdims=False)
    k = k.astype(jnp.bfloat16).reshape(SKV, N_DEV, HEADS_PER_DEV, D_HEAD)
    v = v.astype(jnp.bfloat16).reshape(SKV, N_DEV, HEADS_PER_DEV, D_HEAD)

    w_all = pl.pallas_call(
        _ag_body,
        out_shape=jax.ShapeDtypeStruct((N_DEV, 2, D_MODEL, D_MODEL),
                                       jnp.bfloat16),
        in_specs=[pl.BlockSpec(memory_space=pltpu.VMEM)],
        out_specs=pl.BlockSpec(memory_space=pltpu.VMEM),
        scratch_shapes=[
            pltpu.SemaphoreType.DMA((N_DEV - 1,)),
            pltpu.SemaphoreType.DMA((N_DEV - 1,)),
        ],
        compiler_params=pltpu.CompilerParams(collective_id=0),
    )(w_cat)

    out = pl.pallas_call(
        _attn_body,
        grid=(N_DEV,),
        out_shape=jax.ShapeDtypeStruct((SQ, D_MODEL), jnp.float32),
        in_specs=[
            pl.BlockSpec((SQ, D_MODEL), lambda gg: (0, 0)),
            pl.BlockSpec((1, 2, D_MODEL, D_MODEL),
                         lambda gg: (gg, 0, 0, 0)),
            pl.BlockSpec(memory_space=pltpu.MemorySpace.HBM),
            pl.BlockSpec(memory_space=pltpu.MemorySpace.HBM),
        ],
        out_specs=pl.BlockSpec((SQ, D_MODEL), lambda gg: (0, 0)),
        scratch_shapes=[
            pltpu.VMEM((SQ, D_MODEL), jnp.bfloat16),
            pltpu.VMEM((SQ, D_MODEL), jnp.bfloat16),
            pltpu.VMEM((SQ, SKV), jnp.bfloat16),
            pltpu.VMEM((SKV, HEADS_PER_DEV, D_HEAD), jnp.bfloat16),
            pltpu.VMEM((SKV, HEADS_PER_DEV, D_HEAD), jnp.bfloat16),
            pltpu.SemaphoreType.DMA,
            pltpu.SemaphoreType.DMA,
        ],
        compiler_params=pltpu.CompilerParams(
            dimension_semantics=("arbitrary",),
            vmem_limit_bytes=62 * 1024 * 1024,
        ),
    )(xb, w_all, k, v)
    return out[None].astype(jnp.float32)


# device time: 1026912 ns/iter; 1.2296x vs baseline; 1.2296x over previous
import jax
import jax.numpy as jnp
from jax import lax
from jax.experimental import pallas as pl
from jax.experimental.pallas import tpu as pltpu

N_DEV = 4
SQ = 2048
SKV = 2048
D_MODEL = 1024
N_HEADS = 32
HEADS_PER_DEV = 8
D_HEAD = 128
BLK = 64
SCALE = 0.08838834764831843
QT = 256
N_QT = SQ // QT
NEG = -1e9


def _ag_body(w_ref, out_ref, send_sems, recv_sems):
    my = lax.axis_index("i")
    left = lax.rem(my + N_DEV - 1, N_DEV)
    right = lax.rem(my + 1, N_DEV)

    barrier = pltpu.get_barrier_semaphore()
    for nbr in (left, right):
        pl.semaphore_signal(barrier, inc=1, device_id=(nbr,),
                            device_id_type=pl.DeviceIdType.MESH)
    pl.semaphore_wait(barrier, 2)

    out_ref[my] = w_ref[...]

    for h in range(N_DEV - 1):
        slot = lax.rem(my - h + N_DEV, N_DEV)
        rdma = pltpu.make_async_remote_copy(
            src_ref=out_ref.at[slot],
            dst_ref=out_ref.at[slot],
            send_sem=send_sems.at[h],
            recv_sem=recv_sems.at[h],
            device_id=(right,),
            device_id_type=pl.DeviceIdType.MESH,
        )
        rdma.start()
        rdma.wait()


def _attn_body(x_ref, w_ref, k_hbm, v_hbm, out_ref,
               q_ref, ctx_ref, bias_ref, kbuf, vbuf, ksem, vsem):
    g = pl.program_id(0)

    kc = pltpu.make_async_copy(k_hbm.at[:, g, :, :], kbuf, ksem)
    vc = pltpu.make_async_copy(v_hbm.at[:, g, :, :], vbuf, vsem)
    kc.start()
    vc.start()

    @pl.when(g == 0)
    def _():
        for t in range(N_QT):
            qi = lax.broadcasted_iota(jnp.int32, (QT, SKV), 0) + t * QT
            kj = lax.broadcasted_iota(jnp.int32, (QT, SKV), 1)
            qb = qi // BLK
            kb = kj // BLK
            keep = (qb == kb) | (kb == 0) | (lax.rem(qb + kb, 3) == 0)
            bias_ref[pl.ds(t * QT, QT), :] = jnp.where(
                keep, 0.0, NEG).astype(jnp.bfloat16)
        out_ref[...] = jnp.zeros((SQ, D_MODEL), jnp.float32)

    wq = w_ref[0, 0]

    def _qproj(t, _):
        r0 = t * QT
        q_ref[pl.ds(r0, QT), :] = jnp.dot(
            x_ref[pl.ds(r0, QT), :], wq,
            preferred_element_type=jnp.float32).astype(jnp.bfloat16)
        return 0

    lax.fori_loop(0, N_QT, _qproj, 0)

    kc.wait()
    vc.wait()

    def _tile(t, _):
        r0 = t * QT
        bias_t = bias_ref[pl.ds(r0, QT), :].astype(jnp.float32)
        for hl in range(HEADS_PER_DEV):
            kh = kbuf[:, hl, :]
            vh = vbuf[:, hl, :]
            col = hl * D_HEAD
            qt = q_ref[pl.ds(r0, QT), col:col + D_HEAD]
            s = lax.dot_general(qt, kh, (((1,), (1,)), ((), ())),
                                preferred_element_type=jnp.float32)
            s = s * SCALE + bias_t
            mx = jnp.max(s, axis=1, keepdims=True)
            e = jnp.exp(s - mx)
            denom = jnp.sum(e, axis=1, keepdims=True)
            wgt = (e / denom).astype(jnp.bfloat16)
            ctx_t = lax.dot_general(wgt, vh, (((1,), (0,)), ((), ())),
                                    preferred_element_type=jnp.float32)
            ctx_ref[pl.ds(r0, QT), col:col + D_HEAD] = ctx_t.astype(
                jnp.bfloat16)
        return 0

    lax.fori_loop(0, N_QT, _tile, 0)

    wo = w_ref[0, 1]

    def _oproj(t, _):
        r0 = t * QT
        out_ref[pl.ds(r0, QT), :] += jnp.dot(
            ctx_ref[pl.ds(r0, QT), :], wo,
            preferred_element_type=jnp.float32)
        return 0

    lax.fori_loop(0, N_QT, _oproj, 0)


def kernel(x, Wq, K_ext, V_ext, Wo):
    my = lax.axis_index("i")
    xb = x[0].astype(jnp.bfloat16)
    w_cat = jnp.stack([Wq.astype(jnp.bfloat16),
                       Wo.astype(jnp.bfloat16)])
    k = lax.dynamic_index_in_dim(K_ext, my, axis=0, keepdims=False)
    v = lax.dynamic_index_in_dim(V_ext, my, axis=0, keepdims=False)
    k = k.astype(jnp.bfloat16).reshape(SKV, N_DEV, HEADS_PER_DEV, D_HEAD)
    v = v.astype(jnp.bfloat16).reshape(SKV, N_DEV, HEADS_PER_DEV, D_HEAD)

    w_all = pl.pallas_call(
        _ag_body,
        out_shape=jax.ShapeDtypeStruct((N_DEV, 2, D_MODEL, D_MODEL),
                                       jnp.bfloat16),
        in_specs=[pl.BlockSpec(memory_space=pltpu.VMEM)],
        out_specs=pl.BlockSpec(memory_space=pltpu.VMEM),
        scratch_shapes=[
            pltpu.SemaphoreType.DMA((N_DEV - 1,)),
            pltpu.SemaphoreType.DMA((N_DEV - 1,)),
        ],
        compiler_params=pltpu.CompilerParams(collective_id=0),
    )(w_cat)

    out = pl.pallas_call(
        _attn_body,
        grid=(N_DEV,),
        out_shape=jax.ShapeDtypeStruct((SQ, D_MODEL), jnp.float32),
        in_specs=[
            pl.BlockSpec((SQ, D_MODEL), lambda gg: (0, 0)),
            pl.BlockSpec((1, 2, D_MODEL, D_MODEL),
                         lambda gg: (gg, 0, 0, 0)),
            pl.BlockSpec(memory_space=pltpu.MemorySpace.HBM),
            pl.BlockSpec(memory_space=pltpu.MemorySpace.HBM),
        ],
        out_specs=pl.BlockSpec((SQ, D_MODEL), lambda gg: (0, 0)),
        scratch_shapes=[
            pltpu.VMEM((SQ, D_MODEL), jnp.bfloat16),
            pltpu.VMEM((SQ, D_MODEL), jnp.bfloat16),
            pltpu.VMEM((SQ, SKV), jnp.bfloat16),
            pltpu.VMEM((SKV, HEADS_PER_DEV, D_HEAD), jnp.bfloat16),
            pltpu.VMEM((SKV, HEADS_PER_DEV, D_HEAD), jnp.bfloat16),
            pltpu.SemaphoreType.DMA,
            pltpu.SemaphoreType.DMA,
        ],
        compiler_params=pltpu.CompilerParams(
            dimension_semantics=("arbitrary",),
            vmem_limit_bytes=62 * 1024 * 1024,
        ),
    )(xb, w_all, k, v)
    return out[None].astype(jnp.float32)


# device time: 543425 ns/iter; 2.3235x vs baseline; 1.8897x over previous
import jax
import jax.numpy as jnp
from jax import lax
from jax.experimental import pallas as pl
from jax.experimental.pallas import tpu as pltpu

N_DEV = 4
SQ = 2048
SKV = 2048
D_MODEL = 1024
N_HEADS = 32
HEADS_PER_DEV = 8
D_HEAD = 128
BLK = 64
SCALE = 0.08838834764831843
QT = 512
N_QT = SQ // QT
NEG = -1e9


def _ag_body(w_ref, out_ref, send_sems, recv_sems):
    my = lax.axis_index("i")
    left = lax.rem(my + N_DEV - 1, N_DEV)
    right = lax.rem(my + 1, N_DEV)

    barrier = pltpu.get_barrier_semaphore()
    for nbr in (left, right):
        pl.semaphore_signal(barrier, inc=1, device_id=(nbr,),
                            device_id_type=pl.DeviceIdType.MESH)
    pl.semaphore_wait(barrier, 2)

    out_ref[my] = w_ref[...]

    for h in range(N_DEV - 1):
        slot = lax.rem(my - h + N_DEV, N_DEV)
        rdma = pltpu.make_async_remote_copy(
            src_ref=out_ref.at[slot],
            dst_ref=out_ref.at[slot],
            send_sem=send_sems.at[h],
            recv_sem=recv_sems.at[h],
            device_id=(right,),
            device_id_type=pl.DeviceIdType.MESH,
        )
        rdma.start()
        rdma.wait()


def _attn_body(x_ref, w_ref, k_ref, v_ref, out_ref, q_ref, ctx_ref, bias_ref):
    h = pl.program_id(0)
    hl = lax.rem(h, HEADS_PER_DEV)

    @pl.when(h == 0)
    def _():
        for t in range(N_QT):
            qi = lax.broadcasted_iota(jnp.int32, (QT, SKV), 0) + t * QT
            kj = lax.broadcasted_iota(jnp.int32, (QT, SKV), 1)
            qb = qi // BLK
            kb = kj // BLK
            keep = (qb == kb) | (kb == 0) | (lax.rem(qb + kb, 3) == 0)
            bias_ref[pl.ds(t * QT, QT), :] = jnp.where(
                keep, 0.0, NEG).astype(jnp.bfloat16)
        out_ref[...] = jnp.zeros((SQ, D_MODEL), jnp.float32)

    @pl.when(hl == 0)
    def _():
        wq = w_ref[0, 0]
        for t in range(N_QT):
            q_ref[pl.ds(t * QT, QT), :] = jnp.dot(
                x_ref[pl.ds(t * QT, QT), :], wq,
                preferred_element_type=jnp.float32).astype(jnp.bfloat16)

    kh = k_ref[0]
    vh = v_ref[0]
    col = hl * D_HEAD
    for t in range(N_QT):
        qt = q_ref[pl.ds(t * QT, QT), pl.ds(col, D_HEAD)]
        s = lax.dot_general(qt, kh, (((1,), (1,)), ((), ())),
                            preferred_element_type=jnp.float32)
        s = s * SCALE + bias_ref[pl.ds(t * QT, QT), :].astype(jnp.float32)
        mx = jnp.max(s, axis=1, keepdims=True)
        e = jnp.exp(s - mx)
        denom = jnp.sum(e, axis=1, keepdims=True)
        wgt = (e / denom).astype(jnp.bfloat16)
        ctx_t = lax.dot_general(wgt, vh, (((1,), (0,)), ((), ())),
                                preferred_element_type=jnp.float32)
        ctx_ref[pl.ds(t * QT, QT), pl.ds(col, D_HEAD)] = ctx_t.astype(
            jnp.bfloat16)

    @pl.when(hl == HEADS_PER_DEV - 1)
    def _():
        wo = w_ref[0, 1]
        for t in range(N_QT):
            out_ref[pl.ds(t * QT, QT), :] += jnp.dot(
                ctx_ref[pl.ds(t * QT, QT), :], wo,
                preferred_element_type=jnp.float32)


def kernel(x, Wq, K_ext, V_ext, Wo):
    my = lax.axis_index("i")
    xb = x[0].astype(jnp.bfloat16)
    w_cat = jnp.stack([Wq.astype(jnp.bfloat16),
                       Wo.astype(jnp.bfloat16)])
    k = lax.dynamic_index_in_dim(K_ext, my, axis=0, keepdims=False)
    v = lax.dynamic_index_in_dim(V_ext, my, axis=0, keepdims=False)
    k = jnp.swapaxes(k.astype(jnp.bfloat16), 0, 1)
    v = jnp.swapaxes(v.astype(jnp.bfloat16), 0, 1)

    w_all = pl.pallas_call(
        _ag_body,
        out_shape=jax.ShapeDtypeStruct((N_DEV, 2, D_MODEL, D_MODEL),
                                       jnp.bfloat16),
        in_specs=[pl.BlockSpec(memory_space=pltpu.VMEM)],
        out_specs=pl.BlockSpec(memory_space=pltpu.VMEM),
        scratch_shapes=[
            pltpu.SemaphoreType.DMA((N_DEV - 1,)),
            pltpu.SemaphoreType.DMA((N_DEV - 1,)),
        ],
        compiler_params=pltpu.CompilerParams(collective_id=0),
    )(w_cat)

    out = pl.pallas_call(
        _attn_body,
        grid=(N_HEADS,),
        out_shape=jax.ShapeDtypeStruct((SQ, D_MODEL), jnp.float32),
        in_specs=[
            pl.BlockSpec((SQ, D_MODEL), lambda hh: (0, 0)),
            pl.BlockSpec((1, 2, D_MODEL, D_MODEL),
                         lambda hh: (hh // HEADS_PER_DEV, 0, 0, 0)),
            pl.BlockSpec((1, SKV, D_HEAD), lambda hh: (hh, 0, 0)),
            pl.BlockSpec((1, SKV, D_HEAD), lambda hh: (hh, 0, 0)),
        ],
        out_specs=pl.BlockSpec((SQ, D_MODEL), lambda hh: (0, 0)),
        scratch_shapes=[
            pltpu.VMEM((SQ, D_MODEL), jnp.bfloat16),
            pltpu.VMEM((SQ, D_MODEL), jnp.bfloat16),
            pltpu.VMEM((SQ, SKV), jnp.bfloat16),
        ],
        compiler_params=pltpu.CompilerParams(
            dimension_semantics=("arbitrary",),
            vmem_limit_bytes=62 * 1024 * 1024,
        ),
    )(xb, w_all, k, v)
    return out[None].astype(jnp.float32)
